# baseline (speedup 1.0000x reference)
import functools
import numpy as np
import jax
import jax.numpy as jnp
from jax.experimental import pallas as pl
from jax.experimental.pallas import tpu as pltpu

N = 10000
E = 160000
F_IN = 256
HID = 512
C = 40
K = 10

ROW_BLK = 1000


def _jackson_damp(K):
    k = np.arange(K + 1, dtype=np.float64)
    c = np.pi / (K + 1)
    damp = ((K - k + 1) * np.cos(k * c) + np.sin(k * c) / np.tan(c)) / (K + 1)
    return jnp.asarray(damp, dtype=jnp.float32)


def _mlp_body(x_ref, w1_ref, b1_ref, w2_ref, b2_ref, h_ref):
    a = jnp.dot(x_ref[...], w1_ref[...], preferred_element_type=jnp.float32)
    a = a + b1_ref[...]
    a = a * jax.nn.sigmoid(a)
    b = jnp.dot(a, w2_ref[...], preferred_element_type=jnp.float32)
    b = b + b2_ref[...]
    h_ref[...] = b * jax.nn.sigmoid(b)


def _mlp(x, W1, b1, W2, b2):
    grid = (N // ROW_BLK,)
    return pl.pallas_call(
        _mlp_body,
        grid=grid,
        in_specs=[
            pl.BlockSpec((ROW_BLK, F_IN), lambda i: (i, 0)),
            pl.BlockSpec((F_IN, HID), lambda i: (0, 0)),
            pl.BlockSpec((1, HID), lambda i: (0, 0)),
            pl.BlockSpec((HID, C), lambda i: (0, 0)),
            pl.BlockSpec((1, C), lambda i: (0, 0)),
        ],
        out_specs=pl.BlockSpec((ROW_BLK, C), lambda i: (i, 0)),
        out_shape=jax.ShapeDtypeStruct((N, C), jnp.float32),
    )(x, W1, b1[None, :], W2, b2[None, :])


def _logsoftmax_body(o_ref, out_ref):
    v = o_ref[...]
    m = jnp.max(v, axis=1, keepdims=True)
    s = jnp.sum(jnp.exp(v - m), axis=1, keepdims=True)
    out_ref[...] = v - m - jnp.log(s)


def _logsoftmax(o):
    grid = (N // ROW_BLK,)
    return pl.pallas_call(
        _logsoftmax_body,
        grid=grid,
        in_specs=[pl.BlockSpec((ROW_BLK, C), lambda i: (i, 0))],
        out_specs=pl.BlockSpec((ROW_BLK, C), lambda i: (i, 0)),
        out_shape=jax.ShapeDtypeStruct((N, C), jnp.float32),
    )(o)


def kernel(x, edge_index, edge_weight, W1, b1, W2, b2):
    h = _mlp(x, W1, b1, W2, b2)

    src = edge_index[0]
    dst = edge_index[1]
    deg = jax.ops.segment_sum(edge_weight, dst, num_segments=N)
    deg_safe = jnp.where(deg > 0, deg, 1.0)
    dinv = jnp.where(deg > 0, jax.lax.rsqrt(deg_safe), 0.0)
    wn = dinv[src] * edge_weight * dinv[dst]
    damp = _jackson_damp(K)

    def prop(v):
        return jax.ops.segment_sum(wn[:, None] * v[src], dst, num_segments=N)

    Tx_0 = h
    out = Tx_0
    Tx_1 = prop(h)
    out = out + damp[1] * Tx_1
    for k in range(2, K + 1):
        Tx_2 = 2.0 * prop(Tx_1) - Tx_0
        out = out + damp[k] * Tx_2
        Tx_0, Tx_1 = Tx_1, Tx_2

    return _logsoftmax(out)
